# baseline (speedup 1.0000x reference)
import functools

import jax
import jax.numpy as jnp
from jax import lax
from jax.experimental import pallas as pl
from jax.experimental.pallas import tpu as pltpu
from jax.experimental.pallas import tpu_sc as plsc

_NC, _NS, _LANES = 2, 16, 16
_NW = _NC * _NS
_G = 8
_NBUF = 2


def _batcher_pairs(n2):
    pairs = []
    p = 1
    while p < n2:
        k = p
        while k >= 1:
            for j in range(k % p, n2 - k, 2 * k):
                for i in range(0, k):
                    if j + i + k < n2 and (i + j) // (p * 2) == (i + j + k) // (p * 2):
                        pairs.append((j + i, j + i + k))
            k //= 2
        p *= 2
    return pairs


def _select_net(n, outs):
    pairs = _batcher_pairs(n)
    needed = set(outs)
    ops = []
    for (i, j) in reversed(pairs):
        ni, nj = i in needed, j in needed
        if not (ni or nj):
            continue
        ops.append(("x" if (ni and nj) else ("n" if ni else "m"), i, j))
        needed.add(i)
        needed.add(j)
    ops.reverse()
    return ops


_SEL16_OPS = _select_net(16, {7, 8})


def _median17(self_val, nbrs):
    w = list(nbrs)
    for kind, i, j in _SEL16_OPS:
        a, b = w[i], w[j]
        if kind == "x":
            w[i] = jnp.minimum(a, b)
            w[j] = jnp.maximum(a, b)
        elif kind == "n":
            w[i] = jnp.minimum(a, b)
        else:
            w[j] = jnp.maximum(a, b)
    return jnp.maximum(w[7], jnp.minimum(self_val, w[8]))


def _mm_body(x_ref, w_ref, o_ref):
    o_ref[...] = jnp.dot(x_ref[...], w_ref[...],
                         preferred_element_type=jnp.float32
                         ).astype(jnp.bfloat16)


def _matmul_bf16(x, w, blk=512):
    n, d = x.shape
    f = w.shape[1]
    return pl.pallas_call(
        _mm_body,
        grid=(n // blk,),
        in_specs=[
            pl.BlockSpec((blk, d), lambda i: (i, 0)),
            pl.BlockSpec((d, f), lambda i: (0, 0)),
        ],
        out_specs=pl.BlockSpec((blk, f), lambda i: (i, 0)),
        out_shape=jax.ShapeDtypeStruct((n, f), jnp.bfloat16),
    )(x, w)


def _make_sc_median(np_, deg, fw, relu):
    npw = np_ // _NW
    ngrp = npw // _G
    nch = fw // _LANES

    mesh = plsc.VectorSubcoreMesh(core_axis_name="c", subcore_axis_name="s",
                                  num_cores=_NC, num_subcores=_NS)

    @functools.partial(
        pl.kernel,
        mesh=mesh,
        out_type=jax.ShapeDtypeStruct((np_, fw), jnp.int32),
        scratch_types=[
            pltpu.VMEM((npw * deg,), jnp.int32),
            pltpu.VMEM((2 * _NBUF, _G * deg // 2, fw), jnp.int32),
            pltpu.VMEM((2 * _NBUF, _G // 2, fw), jnp.int32),
            pltpu.VMEM((_NBUF, _G, fw), jnp.int32),
            pltpu.VMEM_SHARED((np_, fw), jnp.int32),
        ] + [pltpu.SemaphoreType.DMA] * (5 * _NBUF),
    )
    def sc_median(table_hbm, adj_hbm, out_hbm, idx_v, rows_v, self_v, out_v,
                  shared_v, *sems):
        gsem = sems[0:2 * _NBUF]
        ssem = sems[2 * _NBUF:4 * _NBUF]
        osem = sems[4 * _NBUF:5 * _NBUF]
        c = lax.axis_index("c")
        s = lax.axis_index("s")
        wid = s * _NC + c
        base = wid * npw

        pltpu.sync_copy(adj_hbm.at[pl.ds(base * deg, npw * deg)], idx_v)

        stripe = np_ // _NS
        pltpu.sync_copy(table_hbm.at[pl.ds(s * stripe, stripe)],
                        shared_v.at[pl.ds(s * stripe, stripe)])
        plsc.subcore_barrier()

        hg = _G // 2
        nsub = 2 * _NBUF

        def issue_gather(u, b):
            node0 = base + u * hg
            pltpu.async_copy(
                shared_v.at[idx_v.at[pl.ds(u * (hg * deg), hg * deg)]],
                rows_v.at[b], gsem[b])
            pltpu.async_copy(shared_v.at[pl.ds(node0, hg)],
                             self_v.at[b], ssem[b])

        def wait_gather(u, b):
            node0 = base + u * hg
            pltpu.make_async_copy(
                shared_v.at[idx_v.at[pl.ds(u * (hg * deg), hg * deg)]],
                rows_v.at[b], gsem[b]).wait()
            pltpu.make_async_copy(shared_v.at[pl.ds(node0, hg)],
                                  self_v.at[b], ssem[b]).wait()

        def issue_out(g, b):
            node0 = base + g * _G
            pltpu.async_copy(out_v.at[b],
                             out_hbm.at[pl.ds(node0, _G)], osem[b])

        def wait_out(g, b):
            node0 = base + g * _G
            pltpu.make_async_copy(out_v.at[b],
                                  out_hbm.at[pl.ds(node0, _G)],
                                  osem[b]).wait()

        def compute(b, bo, h):
            def f32_even(v):
                return lax.bitcast_convert_type(lax.shift_left(v, 16),
                                                jnp.float32)

            def f32_odd(v):
                return lax.bitcast_convert_type(v, jnp.float32)

            def med_body(t, carry):
                i = t // nch
                w0 = (t % nch) * _LANES
                sv = self_v[b, i, pl.ds(w0, _LANES)]
                vs = [rows_v[b, i * deg + k, pl.ds(w0, _LANES)]
                      for k in range(deg)]
                med_e = _median17(f32_even(sv), [f32_even(v) for v in vs])
                med_o = _median17(f32_odd(sv), [f32_odd(v) for v in vs])
                if relu:
                    med_e = jnp.maximum(med_e, 0.0)
                    med_o = jnp.maximum(med_o, 0.0)
                ei = lax.bitcast_convert_type(med_e, jnp.int32)
                oi = lax.bitcast_convert_type(med_o, jnp.int32)
                out_v[bo, h * hg + i, pl.ds(w0, _LANES)] = (
                    lax.shift_right_logical(ei, 16)
                    | (oi & jnp.int32(-65536)))
                return carry

            lax.fori_loop(0, hg * nch, med_body, 0)

        nu = 2 * ngrp

        for b in range(nsub):
            issue_gather(jnp.int32(b), b)

        def outer(step, carry):
            for j in range(_NBUF):
                g = step * _NBUF + j
                bo = j

                @pl.when(g >= _NBUF)
                def _():
                    wait_out(g - _NBUF, bo)

                for h in range(2):
                    u = 2 * g + h
                    b = (2 * j + h) % nsub
                    wait_gather(u, b)
                    compute(b, bo, h)

                    @pl.when(u + nsub < nu)
                    def _():
                        issue_gather(u + nsub, b)

                issue_out(g, bo)

            return carry

        lax.fori_loop(0, ngrp // _NBUF, outer, 0)

        for g in range(ngrp - _NBUF, ngrp):
            wait_out(jnp.int32(g), g % _NBUF)

    return sc_median


def _pack_i32(a):
    n, f = a.shape
    return lax.bitcast_convert_type(a.reshape(n, f // 2, 2), jnp.int32)


def _unpack_bf16(a):
    n, fw = a.shape
    return lax.bitcast_convert_type(a, jnp.bfloat16).reshape(n, 2 * fw)


@jax.jit
def kernel(x, adj, W1, W2):
    n, d = x.shape
    deg = adj.shape[1]
    f1 = W1.shape[1]
    f2 = W2.shape[1]

    align = _NW * _G
    np_ = ((n + align - 1) // align) * align

    x_pad = jnp.pad(x, ((0, np_ - n), (0, 0)))
    adj_flat = jnp.pad(adj, ((0, np_ - n), (0, 0))).reshape(-1)

    sc_relu = _make_sc_median(np_, deg, f1 // 2, relu=True)
    sc_plain = _make_sc_median(np_, deg, f2 // 2, relu=False)

    h = _matmul_bf16(x_pad, W1)
    m1 = sc_relu(_pack_i32(h), adj_flat)
    h2 = _matmul_bf16(_unpack_bf16(m1), W2)
    m2 = sc_plain(_pack_i32(h2), adj_flat)
    return _unpack_bf16(m2)[:n].astype(jnp.float32)


# speedup vs baseline: 1.0024x; 1.0024x over previous
import functools

import jax
import jax.numpy as jnp
from jax import lax
from jax.experimental import pallas as pl
from jax.experimental.pallas import tpu as pltpu
from jax.experimental.pallas import tpu_sc as plsc

_NC, _NS, _LANES = 2, 16, 16
_NW = _NC * _NS
_G = 8
_NBUF = 2


def _batcher_pairs(n2):
    pairs = []
    p = 1
    while p < n2:
        k = p
        while k >= 1:
            for j in range(k % p, n2 - k, 2 * k):
                for i in range(0, k):
                    if j + i + k < n2 and (i + j) // (p * 2) == (i + j + k) // (p * 2):
                        pairs.append((j + i, j + i + k))
            k //= 2
        p *= 2
    return pairs


def _select_net(n, outs):
    pairs = _batcher_pairs(n)
    needed = set(outs)
    ops = []
    for (i, j) in reversed(pairs):
        ni, nj = i in needed, j in needed
        if not (ni or nj):
            continue
        ops.append(("x" if (ni and nj) else ("n" if ni else "m"), i, j))
        needed.add(i)
        needed.add(j)
    ops.reverse()
    return ops


_SEL16_OPS = _select_net(16, {7, 8})


def _median17(self_val, nbrs):
    w = list(nbrs)
    for kind, i, j in _SEL16_OPS:
        a, b = w[i], w[j]
        if kind == "x":
            w[i] = jnp.minimum(a, b)
            w[j] = jnp.maximum(a, b)
        elif kind == "n":
            w[i] = jnp.minimum(a, b)
        else:
            w[j] = jnp.maximum(a, b)
    return jnp.maximum(w[7], jnp.minimum(self_val, w[8]))


def _mm_body(x_ref, w_ref, o_ref):
    o_ref[...] = jnp.dot(x_ref[...], w_ref[...],
                         preferred_element_type=jnp.float32
                         ).astype(jnp.bfloat16)


def _matmul_bf16(x, w, blk=512):
    n, d = x.shape
    f = w.shape[1]
    return pl.pallas_call(
        _mm_body,
        grid=(n // blk,),
        in_specs=[
            pl.BlockSpec((blk, d), lambda i: (i, 0)),
            pl.BlockSpec((d, f), lambda i: (0, 0)),
        ],
        out_specs=pl.BlockSpec((blk, f), lambda i: (i, 0)),
        out_shape=jax.ShapeDtypeStruct((n, f), jnp.bfloat16),
    )(x, w)


def _make_sc_median(np_, deg, fw, relu):
    npw = np_ // _NW
    ngrp = npw // _G
    nch = fw // _LANES

    mesh = plsc.VectorSubcoreMesh(core_axis_name="c", subcore_axis_name="s",
                                  num_cores=_NC, num_subcores=_NS)

    @functools.partial(
        pl.kernel,
        mesh=mesh,
        out_type=jax.ShapeDtypeStruct((np_, fw), jnp.int32),
        scratch_types=[
            pltpu.VMEM((npw * deg,), jnp.int32),
            pltpu.VMEM((_NBUF, _G * deg, fw), jnp.int32),
            pltpu.VMEM((_NBUF, _G, fw), jnp.int32),
            pltpu.VMEM((_NBUF, _G, fw), jnp.int32),
            pltpu.VMEM_SHARED((np_, fw), jnp.int32),
        ] + [pltpu.SemaphoreType.DMA] * (3 * _NBUF),
    )
    def sc_median(table_hbm, adj_hbm, out_hbm, idx_v, rows_v, self_v, out_v,
                  shared_v, *sems):
        gsem = sems[0:_NBUF]
        ssem = sems[_NBUF:2 * _NBUF]
        osem = sems[2 * _NBUF:3 * _NBUF]
        c = lax.axis_index("c")
        s = lax.axis_index("s")
        wid = s * _NC + c
        base = wid * npw

        pltpu.sync_copy(adj_hbm.at[pl.ds(base * deg, npw * deg)], idx_v)

        stripe = np_ // _NS
        pltpu.sync_copy(table_hbm.at[pl.ds(s * stripe, stripe)],
                        shared_v.at[pl.ds(s * stripe, stripe)])
        plsc.subcore_barrier()

        def issue_gather(g, b):
            node0 = base + g * _G
            pltpu.async_copy(
                shared_v.at[idx_v.at[pl.ds(g * (_G * deg), _G * deg)]],
                rows_v.at[b], gsem[b])
            pltpu.async_copy(shared_v.at[pl.ds(node0, _G)],
                             self_v.at[b], ssem[b])

        def wait_gather(g, b):
            node0 = base + g * _G
            pltpu.make_async_copy(
                shared_v.at[idx_v.at[pl.ds(g * (_G * deg), _G * deg)]],
                rows_v.at[b], gsem[b]).wait()
            pltpu.make_async_copy(shared_v.at[pl.ds(node0, _G)],
                                  self_v.at[b], ssem[b]).wait()

        def issue_out(g, b):
            node0 = base + g * _G
            pltpu.async_copy(out_v.at[b],
                             out_hbm.at[pl.ds(node0, _G)], osem[b])

        def wait_out(g, b):
            node0 = base + g * _G
            pltpu.make_async_copy(out_v.at[b],
                                  out_hbm.at[pl.ds(node0, _G)],
                                  osem[b]).wait()

        def compute(b):
            def f32_even(v):
                return lax.bitcast_convert_type(lax.shift_left(v, 16),
                                                jnp.float32)

            def f32_odd(v):
                return lax.bitcast_convert_type(v, jnp.float32)

            def med_body(t, carry):
                i = t // nch
                w0 = (t % nch) * _LANES
                sv = self_v[b, i, pl.ds(w0, _LANES)]
                vs = [rows_v[b, i * deg + k, pl.ds(w0, _LANES)]
                      for k in range(deg)]
                med_e = _median17(f32_even(sv), [f32_even(v) for v in vs])
                med_o = _median17(f32_odd(sv), [f32_odd(v) for v in vs])
                if relu:
                    med_e = jnp.maximum(med_e, 0.0)
                    med_o = jnp.maximum(med_o, 0.0)
                ei = lax.bitcast_convert_type(med_e, jnp.int32)
                oi = lax.bitcast_convert_type(med_o, jnp.int32)
                out_v[b, i, pl.ds(w0, _LANES)] = (
                    lax.shift_right_logical(ei, 16)
                    | (oi & jnp.int32(-65536)))
                return carry

            lax.fori_loop(0, _G * nch, med_body, 0)

        for b in range(_NBUF):
            issue_gather(jnp.int32(b), b)

        def outer(step, carry):
            for b in range(_NBUF):
                g = step * _NBUF + b
                wait_gather(g, b)

                @pl.when(g >= _NBUF)
                def _():
                    wait_out(g - _NBUF, b)

                compute(b)
                issue_out(g, b)

                @pl.when(g + _NBUF < ngrp)
                def _():
                    issue_gather(g + _NBUF, b)

            return carry

        lax.fori_loop(0, ngrp // _NBUF, outer, 0)

        for g in range((ngrp // _NBUF) * _NBUF, ngrp):
            b = g % _NBUF
            wait_gather(jnp.int32(g), b)
            wait_out(jnp.int32(g - _NBUF), b)
            compute(b)
            issue_out(jnp.int32(g), b)

        for g in range(ngrp - _NBUF, ngrp):
            wait_out(jnp.int32(g), g % _NBUF)

    return sc_median


def _pack_i32(a):
    n, f = a.shape
    return lax.bitcast_convert_type(a.reshape(n, f // 2, 2), jnp.int32)


def _unpack_bf16(a):
    n, fw = a.shape
    return lax.bitcast_convert_type(a, jnp.bfloat16).reshape(n, 2 * fw)


@jax.jit
def kernel(x, adj, W1, W2):
    n, d = x.shape
    deg = adj.shape[1]
    f1 = W1.shape[1]
    f2 = W2.shape[1]

    align = _NW * _G
    np_ = ((n + align - 1) // align) * align

    x_pad = jnp.pad(x, ((0, np_ - n), (0, 0)))
    adj_flat = jnp.pad(adj, ((0, np_ - n), (0, 0))).reshape(-1)

    sc_relu = _make_sc_median(np_, deg, f1 // 2, relu=True)
    sc_plain = _make_sc_median(np_, deg, f2 // 2, relu=False)

    h = _matmul_bf16(x_pad, W1)
    m1 = sc_relu(_pack_i32(h), adj_flat)
    h2 = _matmul_bf16(_unpack_bf16(m1), W2)
    m2 = sc_plain(_pack_i32(h2), adj_flat)
    return _unpack_bf16(m2)[:n].astype(jnp.float32)
